# baseline (speedup 1.0000x reference)
import jax
import jax.numpy as jnp
from jax import lax
from jax.experimental import pallas as pl
from jax.experimental.pallas import tpu as pltpu
from jax.experimental.pallas import tpu_sc as plsc

_B = 4
_S = 8192
_H = 768
_EPS = 1e-12
_NC = 2
_NS = 16
_NW = _NC * _NS
_SPW = _S // _NW
_K = 32
_NPC = _SPW // _K
_L = 16
_NROUND = _NPC * _B


def _nr_rsqrt(v):
    i = lax.bitcast_convert_type(v, jnp.int32)
    y = lax.bitcast_convert_type(
        jnp.int32(0x5F3759DF) - lax.shift_right_arithmetic(i, 1), jnp.float32)
    for _ in range(2):
        y = y * (1.5 - 0.5 * v * y * y)
    return y


def _tree_sum(vs):
    vs = list(vs)
    while len(vs) > 1:
        nxt = [vs[i] + vs[i + 1] for i in range(0, len(vs) - 1, 2)]
        if len(vs) % 2:
            nxt.append(vs[-1])
        vs = nxt
    return vs[0]


def _body(ids_hbm, tok_hbm, pos_hbm, gamma_hbm, beta_hbm, out_hbm,
          idx_all, pos_v, tok_0, tok_1, tok_2, gam_v, bet_v, r_st, nmr_st,
          gsem_0, gsem_1, gsem_2, osem_0, osem_1, osem_2):
    cid = lax.axis_index("c")
    sid = lax.axis_index("s")
    wid = sid * _NC + cid
    s_base = wid * _SPW

    toks = (tok_0, tok_1, tok_2)
    gsems = (gsem_0, gsem_1, gsem_2)
    osems = (osem_0, osem_1, osem_2)

    pltpu.sync_copy(gamma_hbm, gam_v)
    pltpu.sync_copy(beta_hbm, bet_v)
    for b in range(_B):
        pltpu.sync_copy(ids_hbm.at[b, pl.ds(s_base, _SPW)],
                        idx_all.at[pl.ds(b * _SPW, _SPW)])

    def idx_slice(r):
        pc = lax.shift_right_logical(r, 2)
        b = lax.bitwise_and(r, 3)
        return idx_all.at[pl.ds(b * _SPW + pc * _K, _K)]

    def out_slice(r):
        pc = lax.shift_right_logical(r, 2)
        b = lax.bitwise_and(r, 3)
        return out_hbm.at[b, pl.ds(s_base + pc * _K, _K)]

    inv = jnp.float32(1.0 / _H)
    nchunk = _H // _L

    def compute_chunk(tok_buf):
        def p1(r):
            vs = []
            for cc in range(nchunk):
                sl = pl.ds(cc * _L, _L)
                v = tok_buf[r, sl] + pos_v[r, sl]
                tok_buf[r, sl] = v
                vs.append(v)
            s1 = _tree_sum(vs)
            s2 = _tree_sum([v * v for v in vs])
            mv = jnp.full((_L,), jnp.sum(s1)) * inv
            qv = jnp.full((_L,), jnp.sum(s2)) * inv
            rr = _nr_rsqrt(qv - mv * mv + jnp.float32(_EPS))
            r_st[r, :] = rr
            nmr_st[r, :] = -(mv * rr)

        plsc.parallel_loop(0, _K, 1, unroll=2)(p1)

        strip = 16
        for s in range(nchunk // strip):
            gs = [gam_v[pl.ds((s * strip + j) * _L, _L)] for j in range(strip)]
            bs = [bet_v[pl.ds((s * strip + j) * _L, _L)] for j in range(strip)]

            def p2(r, _gs=gs, _bs=bs, _s=s):
                rv = r_st[r, :]
                nv = nmr_st[r, :]
                for j in range(strip):
                    sl = pl.ds((_s * strip + j) * _L, _L)
                    x = tok_buf[r, sl]
                    tok_buf[r, sl] = (x * rv + nv) * _gs[j] + _bs[j]

            plsc.parallel_loop(0, _K, 1, unroll=2)(p2)

    def do_round(r, cur):
        nxt = (cur + 2) % 3

        @pl.when(r < _NROUND)
        def _():
            pc = lax.shift_right_logical(r, 2)
            b = lax.bitwise_and(r, 3)
            s0 = s_base + pc * _K

            @pl.when(b == 0)
            def _():
                pltpu.sync_copy(pos_hbm.at[pl.ds(s0, _K)], pos_v)

            pltpu.make_async_copy(tok_hbm.at[idx_slice(r)], toks[cur],
                                  gsems[cur]).wait()

            @pl.when(r + 2 < _NROUND)
            def _():
                @pl.when(r >= 1)
                def _():
                    pltpu.make_async_copy(toks[nxt], out_slice(r),
                                          osems[nxt]).wait()
                pltpu.async_copy(tok_hbm.at[idx_slice(r + 2)], toks[nxt],
                                 gsems[nxt])

            compute_chunk(toks[cur])
            pltpu.async_copy(toks[cur], out_slice(r), osems[cur])

    pltpu.async_copy(tok_hbm.at[idx_slice(jnp.int32(0))], tok_0, gsem_0)
    pltpu.async_copy(tok_hbm.at[idx_slice(jnp.int32(1))], tok_1, gsem_1)

    def triple(jj, _):
        r0 = jj * 3
        do_round(r0, 0)
        do_round(r0 + 1, 1)
        do_round(r0 + 2, 2)
        return 0

    lax.fori_loop(0, (_NROUND + 2) // 3, triple, 0)

    for k in range(3):
        pltpu.make_async_copy(toks[k], out_hbm.at[0, pl.ds(s_base, _K)],
                              osems[k]).wait()


_mesh = plsc.VectorSubcoreMesh(
    core_axis_name="c", subcore_axis_name="s", num_cores=_NC, num_subcores=_NS)

_embed_ln = pl.kernel(
    _body,
    out_type=jax.ShapeDtypeStruct((_B, _S, _H), jnp.float32),
    mesh=_mesh,
    scratch_types=[
        pltpu.VMEM((_B * _SPW,), jnp.int32),
        pltpu.VMEM((_K, _H), jnp.float32),
        pltpu.VMEM((_K, _H), jnp.float32),
        pltpu.VMEM((_K, _H), jnp.float32),
        pltpu.VMEM((_K, _H), jnp.float32),
        pltpu.VMEM((_H,), jnp.float32),
        pltpu.VMEM((_H,), jnp.float32),
        pltpu.VMEM((_K, _L), jnp.float32),
        pltpu.VMEM((_K, _L), jnp.float32),
        pltpu.SemaphoreType.DMA,
        pltpu.SemaphoreType.DMA,
        pltpu.SemaphoreType.DMA,
        pltpu.SemaphoreType.DMA,
        pltpu.SemaphoreType.DMA,
        pltpu.SemaphoreType.DMA,
    ],
    compiler_params=pltpu.CompilerParams(
        use_tc_tiling_on_sc=True, needs_layout_passes=False),
)


def kernel(input_ids, tok_table, pos_table, gamma, beta):
    return _embed_ln(input_ids.astype(jnp.int32), tok_table, pos_table,
                     gamma, beta)
